# baseline (speedup 1.0000x reference)
import jax
import jax.numpy as jnp
from jax.experimental import pallas as pl
from jax.experimental.pallas import tpu as pltpu

E = 6
TOP_K = 3
B, C, H, W = 2, 192, 224, 224
HW = H * W
NBC = B * C


def _pool_body(x_ref, out_ref):
    xb = x_ref[...]
    mx = jnp.max(xb, axis=1, keepdims=True)
    sm = jnp.sum(xb, axis=1, keepdims=True)
    out_ref[...] = mx + sm * (1.0 / HW)


def _pool(x2):
    rows = 48
    grid = (NBC // rows,)
    return pl.pallas_call(
        _pool_body,
        grid=grid,
        in_specs=[pl.BlockSpec((rows, HW), lambda i: (i, 0))],
        out_specs=pl.BlockSpec((rows, 1), lambda i: (i, 0)),
        out_shape=jax.ShapeDtypeStruct((NBC, 1), jnp.float32),
    )(x2)


def _gate_body(pooled_ref, fc0_w_ref, fc0_b_ref, fc1_w_ref, fc1_b_ref,
               cof_ref, idx_ref):
    pooled = pooled_ref[...]
    dn = (((1,), (1,)), ((), ()))
    g = jax.lax.dot_general(pooled, fc1_w_ref[...], dn,
                            preferred_element_type=jnp.float32)
    g = g + fc1_b_ref[...]
    g = jnp.where(g > 0, g, 0.2 * g)

    z = jax.lax.dot_general(pooled, fc0_w_ref[...], dn,
                            preferred_element_type=jnp.float32)
    z = z + fc0_b_ref[...]
    noise = jnp.maximum(z, 0.0) + jnp.log1p(jnp.exp(-jnp.abs(z)))
    nmean = jnp.sum(noise, axis=1, keepdims=True) * (1.0 / E)
    dev = noise - nmean
    std = jnp.sqrt(jnp.sum(dev * dev, axis=1, keepdims=True) * (1.0 / (E - 1)))
    s = g + dev / std

    lane = jax.lax.broadcasted_iota(jnp.int32, (B, E), 1)
    rank = jnp.zeros((B, E), jnp.int32)
    for j in range(E):
        sj = s[:, j:j + 1]
        beats = (sj > s) | ((sj == s) & (j < lane))
        rank = rank + beats.astype(jnp.int32)
    mask = rank < TOP_K

    neg = jnp.float32(-1e30)
    gm = jnp.where(mask, g, neg)
    m = jnp.max(gm, axis=1, keepdims=True)
    ex = jnp.where(mask, jnp.exp(g - m), 0.0)
    cof_ref[...] = ex / jnp.sum(ex, axis=1, keepdims=True)

    cols = []
    for k in range(TOP_K):
        sel = (rank == k).astype(jnp.int32)
        cols.append(jnp.sum(sel * lane, axis=1, keepdims=True))
    idx_ref[...] = jnp.concatenate(cols, axis=1)


def _gate(pooled, fc0_w, fc0_b, fc1_w, fc1_b):
    return pl.pallas_call(
        _gate_body,
        in_specs=[
            pl.BlockSpec((B, C), lambda: (0, 0)),
            pl.BlockSpec((E, C), lambda: (0, 0)),
            pl.BlockSpec((1, E), lambda: (0, 0)),
            pl.BlockSpec((E, C), lambda: (0, 0)),
            pl.BlockSpec((1, E), lambda: (0, 0)),
        ],
        out_specs=[
            pl.BlockSpec((B, E), lambda: (0, 0)),
            pl.BlockSpec((B, TOP_K), lambda: (0, 0)),
        ],
        out_shape=[
            jax.ShapeDtypeStruct((B, E), jnp.float32),
            jax.ShapeDtypeStruct((B, TOP_K), jnp.int32),
        ],
    )(pooled, fc0_w, fc0_b.reshape(1, E), fc1_w, fc1_b.reshape(1, E))


CB = 32


HP = H + 8
WP = W + 32


def _stencil(v, w, oL=None, oR=None):
    if oL is None:
        oL = jnp.roll(v, 1, axis=1)
        oR = jnp.roll(v, -1, axis=1)
    r0 = w[0] * oL + w[1] * v + w[2] * oR
    r1 = w[3] * oL + w[4] * v + w[5] * oR
    r2 = w[6] * oL + w[7] * v + w[8] * oR
    return jnp.roll(r0, 1, axis=0) + r1 + jnp.roll(r2, -1, axis=0)


def _conv_body(idx_ref, x_ref, ew1_ref, eb1_ref, ew2_ref, eb2_ref, cof_ref,
               out_ref):
    i = pl.program_id(0)
    b = i // (C // CB)

    col = jax.lax.broadcasted_iota(jnp.int32, (HP, WP), 1)
    row = jax.lax.broadcasted_iota(jnp.int32, (HP, WP), 0)
    interior = (row >= 1) & (row <= H) & (col >= 1) & (col <= W)

    for ci in range(CB):
        xv = jax.lax.pad(x_ref[ci], jnp.float32(0.0),
                         ((1, HP - H - 1, 0), (1, WP - W - 1, 0)))
        acc = None
        bsum = jnp.float32(0.0)
        for k in range(TOP_K):
            e = idx_ref[b, k]
            coef = cof_ref[b, e]
            w1 = [ew1_ref[e, ci, 0, t] for t in range(9)]
            b1 = eb1_ref[e, ci, 0, 0]
            h = jnp.maximum(_stencil(xv, w1) + b1, 0.0)
            h = jnp.where(interior, h, 0.0)
            w2 = [coef * ew2_ref[e, ci, 0, t] for t in range(9)]
            bsum = bsum + coef * eb2_ref[e, ci, 0, 0]
            y = _stencil(h, w2)
            acc = y if acc is None else acc + y
        out_ref[ci] = acc[1:H + 1, 1:W + 1] + bsum


def _conv(idx, x3, ew1v, eb1v, ew2v, eb2v, cof):
    def widx(i, idx_ref):
        return (0, i % (C // CB), 0, 0)

    grid_spec = pltpu.PrefetchScalarGridSpec(
        num_scalar_prefetch=1,
        grid=(NBC // CB,),
        in_specs=[
            pl.BlockSpec((CB, H, W), lambda i, idx_ref: (i, 0, 0)),
            pl.BlockSpec((E, CB, 1, 9), widx),
            pl.BlockSpec((E, CB, 1, 1), widx),
            pl.BlockSpec((E, CB, 1, 9), widx),
            pl.BlockSpec((E, CB, 1, 1), widx),
            pl.BlockSpec(memory_space=pltpu.SMEM),
        ],
        out_specs=pl.BlockSpec((CB, H, W), lambda i, idx_ref: (i, 0, 0)),
    )
    return pl.pallas_call(
        _conv_body,
        grid_spec=grid_spec,
        out_shape=jax.ShapeDtypeStruct((NBC, H, W), jnp.float32),
    )(idx, x3, ew1v, eb1v, ew2v, eb2v, cof)


@jax.jit
def kernel(x, fc0_w, fc0_b, fc1_w, fc1_b, ew1, eb1, ew2, eb2):
    x2 = x.reshape(NBC, HW)
    pooled = _pool(x2).reshape(B, C)
    cof, idx = _gate(pooled, fc0_w, fc0_b, fc1_w, fc1_b)

    x3 = x.reshape(NBC, H, W)
    ew1v = ew1.reshape(E, C, 1, 9)
    eb1v = eb1.reshape(E, C, 1, 1)
    ew2v = ew2.reshape(E, C, 1, 9)
    eb2v = eb2.reshape(E, C, 1, 1)
    out = _conv(idx, x3, ew1v, eb1v, ew2v, eb2v, cof)
    return out.reshape(B, C, H, W)


# speedup vs baseline: 1.1959x; 1.1959x over previous
import jax
import jax.numpy as jnp
from jax.experimental import pallas as pl
from jax.experimental.pallas import tpu as pltpu

E = 6
TOP_K = 3
B, C, H, W = 2, 192, 224, 224
HW = H * W
NBC = B * C


def _pool_body(x_ref, out_ref):
    xb = x_ref[...]
    mx = jnp.max(xb, axis=1, keepdims=True)
    sm = jnp.sum(xb, axis=1, keepdims=True)
    out_ref[...] = mx + sm * (1.0 / HW)


def _pool(x2):
    rows = 48
    grid = (NBC // rows,)
    return pl.pallas_call(
        _pool_body,
        grid=grid,
        in_specs=[pl.BlockSpec((rows, HW), lambda i: (i, 0))],
        out_specs=pl.BlockSpec((rows, 1), lambda i: (i, 0)),
        out_shape=jax.ShapeDtypeStruct((NBC, 1), jnp.float32),
    )(x2)


def _gate_body(pooled_ref, fc0_w_ref, fc0_b_ref, fc1_w_ref, fc1_b_ref,
               cof_ref, idx_ref):
    pooled = pooled_ref[...]
    dn = (((1,), (1,)), ((), ()))
    g = jax.lax.dot_general(pooled, fc1_w_ref[...], dn,
                            preferred_element_type=jnp.float32)
    g = g + fc1_b_ref[...]
    g = jnp.where(g > 0, g, 0.2 * g)

    z = jax.lax.dot_general(pooled, fc0_w_ref[...], dn,
                            preferred_element_type=jnp.float32)
    z = z + fc0_b_ref[...]
    noise = jnp.maximum(z, 0.0) + jnp.log1p(jnp.exp(-jnp.abs(z)))
    nmean = jnp.sum(noise, axis=1, keepdims=True) * (1.0 / E)
    dev = noise - nmean
    std = jnp.sqrt(jnp.sum(dev * dev, axis=1, keepdims=True) * (1.0 / (E - 1)))
    s = g + dev / std

    lane = jax.lax.broadcasted_iota(jnp.int32, (B, E), 1)
    rank = jnp.zeros((B, E), jnp.int32)
    for j in range(E):
        sj = s[:, j:j + 1]
        beats = (sj > s) | ((sj == s) & (j < lane))
        rank = rank + beats.astype(jnp.int32)
    mask = rank < TOP_K

    neg = jnp.float32(-1e30)
    gm = jnp.where(mask, g, neg)
    m = jnp.max(gm, axis=1, keepdims=True)
    ex = jnp.where(mask, jnp.exp(g - m), 0.0)
    cof_ref[...] = ex / jnp.sum(ex, axis=1, keepdims=True)

    cols = []
    for k in range(TOP_K):
        sel = (rank == k).astype(jnp.int32)
        cols.append(jnp.sum(sel * lane, axis=1, keepdims=True))
    idx_ref[...] = jnp.concatenate(cols, axis=1)


def _gate(pooled, fc0_w, fc0_b, fc1_w, fc1_b):
    return pl.pallas_call(
        _gate_body,
        in_specs=[
            pl.BlockSpec((B, C), lambda: (0, 0)),
            pl.BlockSpec((E, C), lambda: (0, 0)),
            pl.BlockSpec((1, E), lambda: (0, 0)),
            pl.BlockSpec((E, C), lambda: (0, 0)),
            pl.BlockSpec((1, E), lambda: (0, 0)),
        ],
        out_specs=[
            pl.BlockSpec((B, E), lambda: (0, 0)),
            pl.BlockSpec((B, TOP_K), lambda: (0, 0)),
        ],
        out_shape=[
            jax.ShapeDtypeStruct((B, E), jnp.float32),
            jax.ShapeDtypeStruct((B, TOP_K), jnp.int32),
        ],
    )(pooled, fc0_w, fc0_b.reshape(1, E), fc1_w, fc1_b.reshape(1, E))


CB = 16


HP = H + 8
WP = W + 32


def _stencil(v, w, oL=None, oR=None):
    if oL is None:
        oL = jnp.roll(v, 1, axis=1)
        oR = jnp.roll(v, -1, axis=1)
    r0 = w[0] * oL + w[1] * v + w[2] * oR
    r1 = w[3] * oL + w[4] * v + w[5] * oR
    r2 = w[6] * oL + w[7] * v + w[8] * oR
    return jnp.roll(r0, 1, axis=0) + r1 + jnp.roll(r2, -1, axis=0)


def _conv_body(idx_ref, x_ref, ew1_ref, eb1_ref, ew2_ref, eb2_ref, cof_ref,
               out_ref):
    i = pl.program_id(0)
    b = i // (C // CB)

    col = jax.lax.broadcasted_iota(jnp.int32, (HP, WP), 1)
    row = jax.lax.broadcasted_iota(jnp.int32, (HP, WP), 0)
    interior = (row >= 1) & (row <= H) & (col >= 1) & (col <= W)

    for ci in range(CB):
        xv = jax.lax.pad(x_ref[ci], jnp.float32(0.0),
                         ((1, HP - H - 1, 0), (1, WP - W - 1, 0)))
        acc = None
        bsum = jnp.float32(0.0)
        for k in range(TOP_K):
            e = idx_ref[b, k]
            coef = cof_ref[b, e]
            w1 = [ew1_ref[e, ci, 0, t] for t in range(9)]
            b1 = eb1_ref[e, ci, 0, 0]
            h = jnp.maximum(_stencil(xv, w1) + b1, 0.0)
            h = jnp.where(interior, h, 0.0)
            w2 = [coef * ew2_ref[e, ci, 0, t] for t in range(9)]
            bsum = bsum + coef * eb2_ref[e, ci, 0, 0]
            y = _stencil(h, w2)
            acc = y if acc is None else acc + y
        out_ref[ci] = acc[1:H + 1, 1:W + 1] + bsum


def _conv(idx, x3, ew1v, eb1v, ew2v, eb2v, cof):
    def widx(i, idx_ref):
        return (0, i % (C // CB), 0, 0)

    grid_spec = pltpu.PrefetchScalarGridSpec(
        num_scalar_prefetch=1,
        grid=(NBC // CB,),
        in_specs=[
            pl.BlockSpec((CB, H, W), lambda i, idx_ref: (i, 0, 0)),
            pl.BlockSpec((E, CB, 1, 9), widx),
            pl.BlockSpec((E, CB, 1, 1), widx),
            pl.BlockSpec((E, CB, 1, 9), widx),
            pl.BlockSpec((E, CB, 1, 1), widx),
            pl.BlockSpec(memory_space=pltpu.SMEM),
        ],
        out_specs=pl.BlockSpec((CB, H, W), lambda i, idx_ref: (i, 0, 0)),
    )
    return pl.pallas_call(
        _conv_body,
        grid_spec=grid_spec,
        out_shape=jax.ShapeDtypeStruct((NBC, H, W), jnp.float32),
    )(idx, x3, ew1v, eb1v, ew2v, eb2v, cof)


@jax.jit
def kernel(x, fc0_w, fc0_b, fc1_w, fc1_b, ew1, eb1, ew2, eb2):
    x2 = x.reshape(NBC, HW)
    pooled = _pool(x2).reshape(B, C)
    cof, idx = _gate(pooled, fc0_w, fc0_b, fc1_w, fc1_b)

    x3 = x.reshape(NBC, H, W)
    ew1v = ew1.reshape(E, C, 1, 9)
    eb1v = eb1.reshape(E, C, 1, 1)
    ew2v = ew2.reshape(E, C, 1, 9)
    eb2v = eb2.reshape(E, C, 1, 1)
    out = _conv(idx, x3, ew1v, eb1v, ew2v, eb2v, cof)
    return out.reshape(B, C, H, W)
